# baseline (speedup 1.0000x reference)
import functools

import jax
import jax.numpy as jnp
from jax import lax
from jax.experimental import pallas as pl
from jax.experimental.pallas import tpu as pltpu
from jax.experimental.pallas import tpu_sc as plsc


def _sc_gather_call(E_player, H_player, E_team, H_team, E_pos, H_pos, E_seq, H_seq):
    B = H_player.shape[0]
    D_id = E_player.shape[1]
    D_pos = E_pos.shape[1]
    D_seq = E_seq.shape[1]

    info = plsc.get_sparse_core_info()
    NC, NS = info.num_cores, info.num_subcores
    NW = NC * NS
    b_per_w = B // NW
    assert B % (8 * NW) == 0

    mesh = plsc.VectorSubcoreMesh(core_axis_name="c", subcore_axis_name="s")

    @functools.partial(
        pl.kernel,
        mesh=mesh,
        out_type=[
            jax.ShapeDtypeStruct((B, D_id), jnp.float32),
            jax.ShapeDtypeStruct((B, D_id), jnp.float32),
            jax.ShapeDtypeStruct((B, D_pos), jnp.float32),
            jax.ShapeDtypeStruct((B, D_seq), jnp.float32),
        ],
        scratch_types=[
            pltpu.VMEM((b_per_w,), jnp.int32),
            pltpu.VMEM((b_per_w,), jnp.int32),
            pltpu.VMEM((b_per_w,), jnp.int32),
            pltpu.VMEM((b_per_w,), jnp.int32),
            pltpu.VMEM((b_per_w, D_id), jnp.float32),
            pltpu.VMEM((b_per_w, D_id), jnp.float32),
            pltpu.VMEM((b_per_w, D_pos), jnp.float32),
            pltpu.VMEM((b_per_w, D_seq), jnp.float32),
            pltpu.SemaphoreType.DMA,
        ],
        compiler_params=pltpu.CompilerParams(use_tc_tiling_on_sc=False),
    )
    def gather_kernel(ep, hp, et, ht, epos, hpos, eseq, hseq,
                      out_p, out_t, out_pos, out_seq,
                      ip, it, ipos, iseq, rp, rt, rpos, rseq, sem):
        wid = lax.axis_index("s") * NC + lax.axis_index("c")
        sl = pl.ds(wid * b_per_w, b_per_w)
        pltpu.sync_copy(hp.at[sl], ip)
        pltpu.sync_copy(ht.at[sl], it)
        pltpu.sync_copy(hpos.at[sl], ipos)
        pltpu.sync_copy(hseq.at[sl], iseq)
        cp = pltpu.async_copy(ep.at[ip], rp, sem)
        ct = pltpu.async_copy(et.at[it], rt, sem)
        cpos = pltpu.async_copy(epos.at[ipos], rpos, sem)
        cseq = pltpu.async_copy(eseq.at[iseq], rseq, sem)
        cp.wait()
        ct.wait()
        cpos.wait()
        cseq.wait()
        pltpu.sync_copy(rp, out_p.at[sl])
        pltpu.sync_copy(rt, out_t.at[sl])
        pltpu.sync_copy(rpos, out_pos.at[sl])
        pltpu.sync_copy(rseq, out_seq.at[sl])

    return gather_kernel(E_player, H_player, E_team, H_team,
                         E_pos, H_pos, E_seq, H_seq)


def _mlp_body(T, D_base, D_model,
              eb, praw, temb, posemb, seqemb, fm,
              w1a, w1p, w1t, w1pos, w1seq, b1, g, lb, w2, b2,
              out, pout):
    f32 = jnp.float32
    keep = 1.0 - fm[...]
    pm = praw[...] * keep
    pout[...] = pm
    hrow = (jnp.dot(pm, w1p[...], preferred_element_type=f32)
            + jnp.dot(temb[...], w1t[...], preferred_element_type=f32)
            + jnp.dot(posemb[...], w1pos[...], preferred_element_type=f32)
            + jnp.dot(seqemb[...], w1seq[...], preferred_element_type=f32)
            + b1[...])
    w1a_v = w1a[...]
    w2_v = w2[...]
    g_v = g[...]
    lb_v = lb[...]
    b2_v = b2[...]
    inv_d = 1.0 / D_model
    for t in range(T):
        x = eb[:, t * D_base:(t + 1) * D_base]
        h = jnp.dot(x, w1a_v, preferred_element_type=f32) + hrow
        h = jnp.maximum(h, 0.0)
        mu = jnp.sum(h, axis=1, keepdims=True) * inv_d
        d = h - mu
        var = jnp.sum(d * d, axis=1, keepdims=True) * inv_d
        hn = d * lax.rsqrt(var + 1e-5) * g_v + lb_v
        out[:, t * D_model:(t + 1) * D_model] = (
            jnp.dot(hn, w2_v, preferred_element_type=f32) + b2_v)


def kernel(e_base, H_player, H_team, H_poshint, H_seqtype, mask_player_vec,
           E_player, E_team, E_pos, E_seq, W1, b1, ln_g, ln_b, W2, b2):
    B, T, D_base = e_base.shape
    D_id = E_player.shape[1]
    D_pos = E_pos.shape[1]
    D_seq = E_seq.shape[1]
    D_model = W2.shape[1]

    player_raw, team_emb, pos_emb, seq_emb = _sc_gather_call(
        E_player, H_player, E_team, H_team,
        E_pos, jnp.maximum(H_poshint, 0), E_seq, H_seqtype)

    o0, o1, o2, o3 = D_base, D_base + D_id, D_base + 2 * D_id, D_base + 2 * D_id + D_pos
    W1a = W1[:o0]
    W1p = W1[o0:o1]
    W1t = W1[o1:o2]
    W1pos = W1[o2:o3]
    W1seq = W1[o3:]

    eb2d = e_base.reshape(B, T * D_base)
    fmask = mask_player_vec.astype(jnp.float32).reshape(B, 1)
    b1r = b1.reshape(1, D_model)
    gr = ln_g.reshape(1, D_model)
    lbr = ln_b.reshape(1, D_model)
    b2r = b2.reshape(1, D_model)

    bB = 128
    nB = B // bB

    row_spec = lambda d: pl.BlockSpec((bB, d), lambda b: (b, 0))
    full_spec = lambda r, c: pl.BlockSpec((r, c), lambda b: (0, 0))

    out2d, player_emb = pl.pallas_call(
        functools.partial(_mlp_body, T, D_base, D_model),
        grid=(nB,),
        in_specs=[
            row_spec(T * D_base),
            row_spec(D_id),
            row_spec(D_id),
            row_spec(D_pos),
            row_spec(D_seq),
            row_spec(1),
            full_spec(D_base, D_model),
            full_spec(D_id, D_model),
            full_spec(D_id, D_model),
            full_spec(D_pos, D_model),
            full_spec(D_seq, D_model),
            full_spec(1, D_model),
            full_spec(1, D_model),
            full_spec(1, D_model),
            full_spec(D_model, D_model),
            full_spec(1, D_model),
        ],
        out_specs=[
            row_spec(T * D_model),
            row_spec(D_id),
        ],
        out_shape=[
            jax.ShapeDtypeStruct((B, T * D_model), jnp.float32),
            jax.ShapeDtypeStruct((B, D_id), jnp.float32),
        ],
        compiler_params=pltpu.CompilerParams(
            dimension_semantics=("arbitrary",),
        ),
    )(eb2d, player_raw, team_emb, pos_emb, seq_emb, fmask,
      W1a, W1p, W1t, W1pos, W1seq, b1r, gr, lbr, W2, b2r)

    return out2d.reshape(B, T, D_model), player_emb, team_emb


# speedup vs baseline: 1.4531x; 1.4531x over previous
import functools

import jax
import jax.numpy as jnp
from jax import lax
from jax.experimental import pallas as pl
from jax.experimental.pallas import tpu as pltpu
from jax.experimental.pallas import tpu_sc as plsc


def _sc_gather_call(E_player, H_player, E_team, H_team, E_pos, H_pos, E_seq, H_seq):
    B = H_player.shape[0]
    D_id = E_player.shape[1]
    D_pos = E_pos.shape[1]
    D_seq = E_seq.shape[1]

    info = plsc.get_sparse_core_info()
    NC, NS, L = info.num_cores, info.num_subcores, info.num_lanes
    NW = NC * NS
    b_per_w = B // NW
    assert B % (8 * NW) == 0

    mesh = plsc.VectorSubcoreMesh(core_axis_name="c", subcore_axis_name="s")

    @functools.partial(
        pl.kernel,
        mesh=mesh,
        out_type=[
            jax.ShapeDtypeStruct((B, D_id), jnp.float32),
            jax.ShapeDtypeStruct((B, D_id), jnp.float32),
            jax.ShapeDtypeStruct((B, D_pos), jnp.float32),
            jax.ShapeDtypeStruct((B, D_seq), jnp.float32),
        ],
        scratch_types=[
            pltpu.VMEM((b_per_w + L,), jnp.int32),
            pltpu.VMEM((b_per_w, D_id), jnp.float32),
            pltpu.VMEM((b_per_w, D_id), jnp.float32),
            pltpu.VMEM((b_per_w, D_pos), jnp.float32),
            pltpu.VMEM((b_per_w, D_seq), jnp.float32),
            pltpu.SemaphoreType.DMA,
        ],
    )
    def gather_kernel(ep, hp, et, ht, epos, hpos, eseq, hseq,
                      out_p, out_t, out_pos, out_seq,
                      iv, rp, rt, rpos, rseq, sem):
        wid = lax.axis_index("s") * NC + lax.axis_index("c")
        sl = pl.ds(wid * b_per_w, b_per_w)

        def enqueue_rows(h_hbm, tab_hbm, rbuf):
            pltpu.sync_copy(h_hbm.at[sl], iv.at[pl.ds(0, b_per_w)])

            def body(i, _):
                idx = iv[pl.ds(i, L)][0]
                pltpu.async_copy(tab_hbm.at[pl.ds(idx, 1)],
                                 rbuf.at[pl.ds(i, 1)], sem)
                return ()

            lax.fori_loop(0, b_per_w, body, ())

        enqueue_rows(hp, ep, rp)
        enqueue_rows(ht, et, rt)
        enqueue_rows(hpos, epos, rpos)
        enqueue_rows(hseq, eseq, rseq)
        pltpu.make_async_copy(ep.at[pl.ds(0, b_per_w)], rp, sem).wait()
        pltpu.make_async_copy(et.at[pl.ds(0, b_per_w)], rt, sem).wait()
        pltpu.make_async_copy(epos.at[pl.ds(0, b_per_w)], rpos, sem).wait()
        pltpu.make_async_copy(eseq.at[pl.ds(0, b_per_w)], rseq, sem).wait()
        pltpu.sync_copy(rp, out_p.at[sl])
        pltpu.sync_copy(rt, out_t.at[sl])
        pltpu.sync_copy(rpos, out_pos.at[sl])
        pltpu.sync_copy(rseq, out_seq.at[sl])

    return gather_kernel(E_player, H_player, E_team, H_team,
                         E_pos, H_pos, E_seq, H_seq)


def _mlp_body(T, D_model,
              eb, praw, temb, posemb, seqemb, fm,
              w1a, w1p, w1t, w1pos, w1seq, b1, g, lb, w2, b2,
              out, pout):
    f32 = jnp.float32
    keep = 1.0 - fm[...]
    pm = praw[...] * keep
    pout[...] = pm
    hrow = (jnp.dot(pm, w1p[...], preferred_element_type=f32)
            + jnp.dot(temb[...], w1t[...], preferred_element_type=f32)
            + jnp.dot(posemb[...], w1pos[...], preferred_element_type=f32)
            + jnp.dot(seqemb[...], w1seq[...], preferred_element_type=f32)
            + b1[...])
    w1a_v = w1a[...]
    w2_v = w2[...]
    g_v = g[...]
    lb_v = lb[...]
    b2_v = b2[...]
    inv_d = 1.0 / D_model
    for t in range(T):
        x = eb[:, t, :]
        h = jnp.dot(x, w1a_v, preferred_element_type=f32) + hrow
        h = jnp.maximum(h, 0.0)
        mu = jnp.sum(h, axis=1, keepdims=True) * inv_d
        d = h - mu
        var = jnp.sum(d * d, axis=1, keepdims=True) * inv_d
        hn = d * lax.rsqrt(var + 1e-5) * g_v + lb_v
        out[:, t, :] = jnp.dot(hn, w2_v, preferred_element_type=f32) + b2_v


def kernel(e_base, H_player, H_team, H_poshint, H_seqtype, mask_player_vec,
           E_player, E_team, E_pos, E_seq, W1, b1, ln_g, ln_b, W2, b2):
    B, T, D_base = e_base.shape
    D_id = E_player.shape[1]
    D_pos = E_pos.shape[1]
    D_seq = E_seq.shape[1]
    D_model = W2.shape[1]

    player_raw, team_emb, pos_emb, seq_emb = _sc_gather_call(
        E_player, H_player, E_team, H_team,
        E_pos, jnp.maximum(H_poshint, 0), E_seq, H_seqtype)

    o0, o1, o2, o3 = D_base, D_base + D_id, D_base + 2 * D_id, D_base + 2 * D_id + D_pos
    W1a = W1[:o0]
    W1p = W1[o0:o1]
    W1t = W1[o1:o2]
    W1pos = W1[o2:o3]
    W1seq = W1[o3:]

    fmask = mask_player_vec.astype(jnp.float32).reshape(B, 1)
    b1r = b1.reshape(1, D_model)
    gr = ln_g.reshape(1, D_model)
    lbr = ln_b.reshape(1, D_model)
    b2r = b2.reshape(1, D_model)

    bB = 256
    nB = B // bB

    row_spec = lambda d: pl.BlockSpec((bB, d), lambda b: (b, 0))
    full_spec = lambda r, c: pl.BlockSpec((r, c), lambda b: (0, 0))

    out3d, player_emb = pl.pallas_call(
        functools.partial(_mlp_body, T, D_model),
        grid=(nB,),
        in_specs=[
            pl.BlockSpec((bB, T, D_base), lambda b: (b, 0, 0)),
            row_spec(D_id),
            row_spec(D_id),
            row_spec(D_pos),
            row_spec(D_seq),
            row_spec(1),
            full_spec(D_base, D_model),
            full_spec(D_id, D_model),
            full_spec(D_id, D_model),
            full_spec(D_pos, D_model),
            full_spec(D_seq, D_model),
            full_spec(1, D_model),
            full_spec(1, D_model),
            full_spec(1, D_model),
            full_spec(D_model, D_model),
            full_spec(1, D_model),
        ],
        out_specs=[
            pl.BlockSpec((bB, T, D_model), lambda b: (b, 0, 0)),
            row_spec(D_id),
        ],
        out_shape=[
            jax.ShapeDtypeStruct((B, T, D_model), jnp.float32),
            jax.ShapeDtypeStruct((B, D_id), jnp.float32),
        ],
        compiler_params=pltpu.CompilerParams(
            dimension_semantics=("parallel",),
        ),
    )(e_base, player_raw, team_emb, pos_emb, seq_emb, fmask,
      W1a, W1p, W1t, W1pos, W1seq, b1r, gr, lbr, W2, b2r)

    return out3d, player_emb, team_emb


# speedup vs baseline: 1.5184x; 1.0449x over previous
import functools

import jax
import jax.numpy as jnp
from jax import lax
from jax.experimental import pallas as pl
from jax.experimental.pallas import tpu as pltpu
from jax.experimental.pallas import tpu_sc as plsc


def _sc_gather_call(E_player, H_player, E_team, H_team, E_pos, H_pos, E_seq, H_seq):
    B = H_player.shape[0]
    D_id = E_player.shape[1]
    D_pos = E_pos.shape[1]
    D_seq = E_seq.shape[1]

    info = plsc.get_sparse_core_info()
    NC, NS, L = info.num_cores, info.num_subcores, info.num_lanes
    NW = NC * NS
    b_per_w = B // NW
    n_chunks = b_per_w // L
    assert B % (L * NW) == 0

    mesh = plsc.VectorSubcoreMesh(core_axis_name="c", subcore_axis_name="s")

    @functools.partial(
        pl.kernel,
        mesh=mesh,
        out_type=[
            jax.ShapeDtypeStruct((B, D_id), jnp.float32),
            jax.ShapeDtypeStruct((B, D_id), jnp.float32),
            jax.ShapeDtypeStruct((B, D_pos), jnp.float32),
            jax.ShapeDtypeStruct((B, D_seq), jnp.float32),
        ],
        scratch_types=[
            pltpu.VMEM((b_per_w,), jnp.int32),
            pltpu.VMEM((b_per_w, D_id), jnp.float32),
            pltpu.VMEM((b_per_w, D_id), jnp.float32),
            pltpu.VMEM((b_per_w, D_pos), jnp.float32),
            pltpu.VMEM((b_per_w, D_seq), jnp.float32),
            pltpu.SemaphoreType.DMA,
        ],
    )
    def gather_kernel(ep, hp, et, ht, epos, hpos, eseq, hseq,
                      out_p, out_t, out_pos, out_seq,
                      iv, rp, rt, rpos, rseq, sem):
        wid = lax.axis_index("s") * NC + lax.axis_index("c")
        sl = pl.ds(wid * b_per_w, b_per_w)

        def enqueue_rows(h_hbm, tab_hbm, rbuf):
            pltpu.sync_copy(h_hbm.at[sl], iv)

            def chunk_body(c, _):
                base = c * L
                chunk = iv[pl.ds(base, L)]
                for j in range(L):
                    pltpu.async_copy(tab_hbm.at[pl.ds(chunk[j], 1)],
                                     rbuf.at[pl.ds(base + j, 1)], sem)
                return ()

            lax.fori_loop(0, n_chunks, chunk_body, ())

        enqueue_rows(hp, ep, rp)
        enqueue_rows(ht, et, rt)
        enqueue_rows(hpos, epos, rpos)
        enqueue_rows(hseq, eseq, rseq)
        pltpu.make_async_copy(ep.at[pl.ds(0, b_per_w)], rp, sem).wait()
        pltpu.make_async_copy(et.at[pl.ds(0, b_per_w)], rt, sem).wait()
        pltpu.make_async_copy(epos.at[pl.ds(0, b_per_w)], rpos, sem).wait()
        pltpu.make_async_copy(eseq.at[pl.ds(0, b_per_w)], rseq, sem).wait()
        pltpu.sync_copy(rp, out_p.at[sl])
        pltpu.sync_copy(rt, out_t.at[sl])
        pltpu.sync_copy(rpos, out_pos.at[sl])
        pltpu.sync_copy(rseq, out_seq.at[sl])

    return gather_kernel(E_player, H_player, E_team, H_team,
                         E_pos, H_pos, E_seq, H_seq)


def _mlp_body(D_model,
              eb, praw, temb, posemb, seqemb, fm,
              w1a, w1p, w1t, w1pos, w1seq, b1, g, lb, w2, b2,
              out, pout, hrow_ref):
    f32 = jnp.float32
    t = pl.program_id(1)

    @pl.when(t == 0)
    def _():
        keep = 1.0 - fm[...]
        pm = praw[...] * keep
        pout[...] = pm
        hrow_ref[...] = (
            jnp.dot(pm, w1p[...], preferred_element_type=f32)
            + jnp.dot(temb[...], w1t[...], preferred_element_type=f32)
            + jnp.dot(posemb[...], w1pos[...], preferred_element_type=f32)
            + jnp.dot(seqemb[...], w1seq[...], preferred_element_type=f32)
            + b1[...])

    x = eb[...]
    h = jnp.dot(x, w1a[...], preferred_element_type=f32) + hrow_ref[...]
    h = jnp.maximum(h, 0.0)
    inv_d = 1.0 / D_model
    mu = jnp.sum(h, axis=1, keepdims=True) * inv_d
    d = h - mu
    var = jnp.sum(d * d, axis=1, keepdims=True) * inv_d
    hn = d * lax.rsqrt(var + 1e-5) * g[...] + lb[...]
    out[...] = jnp.dot(hn, w2[...], preferred_element_type=f32) + b2[...]


def kernel(e_base, H_player, H_team, H_poshint, H_seqtype, mask_player_vec,
           E_player, E_team, E_pos, E_seq, W1, b1, ln_g, ln_b, W2, b2):
    B, T, D_base = e_base.shape
    D_id = E_player.shape[1]
    D_pos = E_pos.shape[1]
    D_seq = E_seq.shape[1]
    D_model = W2.shape[1]

    player_raw, team_emb, pos_emb, seq_emb = _sc_gather_call(
        E_player, H_player, E_team, H_team,
        E_pos, jnp.maximum(H_poshint, 0), E_seq, H_seqtype)

    o0, o1, o2, o3 = D_base, D_base + D_id, D_base + 2 * D_id, D_base + 2 * D_id + D_pos
    W1a = W1[:o0]
    W1p = W1[o0:o1]
    W1t = W1[o1:o2]
    W1pos = W1[o2:o3]
    W1seq = W1[o3:]

    e_bt = jnp.transpose(e_base, (1, 0, 2))
    fmask = mask_player_vec.astype(jnp.float32).reshape(B, 1)
    b1r = b1.reshape(1, D_model)
    gr = ln_g.reshape(1, D_model)
    lbr = ln_b.reshape(1, D_model)
    b2r = b2.reshape(1, D_model)

    bB = 512
    nB = B // bB

    row_spec = lambda d: pl.BlockSpec((bB, d), lambda b, t: (b, 0))
    full_spec = lambda r, c: pl.BlockSpec((r, c), lambda b, t: (0, 0))

    out_tb, player_emb = pl.pallas_call(
        functools.partial(_mlp_body, D_model),
        grid=(nB, T),
        in_specs=[
            pl.BlockSpec((None, bB, D_base), lambda b, t: (t, b, 0)),
            row_spec(D_id),
            row_spec(D_id),
            row_spec(D_pos),
            row_spec(D_seq),
            row_spec(1),
            full_spec(D_base, D_model),
            full_spec(D_id, D_model),
            full_spec(D_id, D_model),
            full_spec(D_pos, D_model),
            full_spec(D_seq, D_model),
            full_spec(1, D_model),
            full_spec(1, D_model),
            full_spec(1, D_model),
            full_spec(D_model, D_model),
            full_spec(1, D_model),
        ],
        out_specs=[
            pl.BlockSpec((None, bB, D_model), lambda b, t: (t, b, 0)),
            row_spec(D_id),
        ],
        out_shape=[
            jax.ShapeDtypeStruct((T, B, D_model), jnp.float32),
            jax.ShapeDtypeStruct((B, D_id), jnp.float32),
        ],
        scratch_shapes=[pltpu.VMEM((bB, D_model), jnp.float32)],
        compiler_params=pltpu.CompilerParams(
            dimension_semantics=("parallel", "arbitrary"),
        ),
    )(e_bt, player_raw, team_emb, pos_emb, seq_emb, fmask,
      W1a, W1p, W1t, W1pos, W1seq, b1r, gr, lbr, W2, b2r)

    return (jnp.transpose(out_tb, (1, 0, 2)), player_emb, team_emb)


# speedup vs baseline: 1.6554x; 1.0903x over previous
import functools

import jax
import jax.numpy as jnp
from jax import lax
from jax.experimental import pallas as pl
from jax.experimental.pallas import tpu as pltpu
from jax.experimental.pallas import tpu_sc as plsc


def _sc_gather_call(E_player, H_player, E_team, H_team, E_pos, H_pos, E_seq, H_seq):
    B = H_player.shape[0]
    D_id = E_player.shape[1]
    D_pos = E_pos.shape[1]
    D_seq = E_seq.shape[1]

    info = plsc.get_sparse_core_info()
    NC, NS, L = info.num_cores, info.num_subcores, info.num_lanes
    NW = NC * NS
    b_per_w = B // NW
    n_chunks = b_per_w // L
    assert B % (L * NW) == 0

    mesh = plsc.VectorSubcoreMesh(core_axis_name="c", subcore_axis_name="s")

    @functools.partial(
        pl.kernel,
        mesh=mesh,
        out_type=[
            jax.ShapeDtypeStruct((B, D_id), jnp.float32),
            jax.ShapeDtypeStruct((B, D_id), jnp.float32),
            jax.ShapeDtypeStruct((B, D_pos), jnp.float32),
            jax.ShapeDtypeStruct((B, D_seq), jnp.float32),
        ],
        scratch_types=[
            pltpu.VMEM((b_per_w,), jnp.int32),
            pltpu.VMEM((b_per_w, D_id), jnp.float32),
            pltpu.VMEM((b_per_w, D_id), jnp.float32),
            pltpu.VMEM((b_per_w, D_pos), jnp.float32),
            pltpu.VMEM((b_per_w, D_seq), jnp.float32),
            pltpu.SemaphoreType.DMA,
        ],
    )
    def gather_kernel(ep, hp, et, ht, epos, hpos, eseq, hseq,
                      out_p, out_t, out_pos, out_seq,
                      iv, rp, rt, rpos, rseq, sem):
        wid = lax.axis_index("s") * NC + lax.axis_index("c")
        sl = pl.ds(wid * b_per_w, b_per_w)

        def enqueue_rows(h_hbm, tab_hbm, rbuf):
            pltpu.sync_copy(h_hbm.at[sl], iv)

            def chunk_body(c, _):
                base = c * L
                chunk = iv[pl.ds(base, L)]
                for j in range(L):
                    pltpu.async_copy(tab_hbm.at[pl.ds(chunk[j], 1)],
                                     rbuf.at[pl.ds(base + j, 1)], sem)
                return ()

            lax.fori_loop(0, n_chunks, chunk_body, ())

        enqueue_rows(hp, ep, rp)
        enqueue_rows(ht, et, rt)
        enqueue_rows(hpos, epos, rpos)
        enqueue_rows(hseq, eseq, rseq)
        pltpu.make_async_copy(ep.at[pl.ds(0, b_per_w)], rp, sem).wait()
        pltpu.make_async_copy(et.at[pl.ds(0, b_per_w)], rt, sem).wait()
        pltpu.make_async_copy(epos.at[pl.ds(0, b_per_w)], rpos, sem).wait()
        pltpu.make_async_copy(eseq.at[pl.ds(0, b_per_w)], rseq, sem).wait()
        pltpu.sync_copy(rp, out_p.at[sl])
        pltpu.sync_copy(rt, out_t.at[sl])
        pltpu.sync_copy(rpos, out_pos.at[sl])
        pltpu.sync_copy(rseq, out_seq.at[sl])

    return gather_kernel(E_player, H_player, E_team, H_team,
                         E_pos, H_pos, E_seq, H_seq)


def _mlp_body(D_model,
              eb, praw, temb, posemb, seqemb, fm,
              w1a, w1p, w1t, w1pos, w1seq, b1, g, lb, w2, b2,
              out, pout, hrow_ref):
    f32 = jnp.float32
    t = pl.program_id(1)

    @pl.when(t == 0)
    def _():
        keep = 1.0 - fm[...]
        pm = praw[...] * keep
        pout[...] = pm
        hrow_ref[...] = (
            jnp.dot(pm, w1p[...], preferred_element_type=f32)
            + jnp.dot(temb[...], w1t[...], preferred_element_type=f32)
            + jnp.dot(posemb[...], w1pos[...], preferred_element_type=f32)
            + jnp.dot(seqemb[...], w1seq[...], preferred_element_type=f32)
            + b1[...])

    bf16 = jnp.bfloat16
    x = eb[...].astype(bf16)
    h = jnp.dot(x, w1a[...].astype(bf16), preferred_element_type=f32) + hrow_ref[...]
    h = jnp.maximum(h, 0.0)
    inv_d = 1.0 / D_model
    mu = jnp.sum(h, axis=1, keepdims=True) * inv_d
    d = h - mu
    var = jnp.sum(d * d, axis=1, keepdims=True) * inv_d
    hn = d * lax.rsqrt(var + 1e-5) * g[...] + lb[...]
    out[...] = jnp.dot(hn.astype(bf16), w2[...].astype(bf16),
                       preferred_element_type=f32) + b2[...]


def kernel(e_base, H_player, H_team, H_poshint, H_seqtype, mask_player_vec,
           E_player, E_team, E_pos, E_seq, W1, b1, ln_g, ln_b, W2, b2):
    B, T, D_base = e_base.shape
    D_id = E_player.shape[1]
    D_pos = E_pos.shape[1]
    D_seq = E_seq.shape[1]
    D_model = W2.shape[1]

    player_raw, team_emb, pos_emb, seq_emb = _sc_gather_call(
        E_player, H_player, E_team, H_team,
        E_pos, jnp.maximum(H_poshint, 0), E_seq, H_seqtype)

    o0, o1, o2, o3 = D_base, D_base + D_id, D_base + 2 * D_id, D_base + 2 * D_id + D_pos
    W1a = W1[:o0]
    W1p = W1[o0:o1]
    W1t = W1[o1:o2]
    W1pos = W1[o2:o3]
    W1seq = W1[o3:]

    e_bt = jnp.transpose(e_base, (1, 0, 2))
    fmask = mask_player_vec.astype(jnp.float32).reshape(B, 1)
    b1r = b1.reshape(1, D_model)
    gr = ln_g.reshape(1, D_model)
    lbr = ln_b.reshape(1, D_model)
    b2r = b2.reshape(1, D_model)

    bB = 1024
    nB = B // bB

    row_spec = lambda d: pl.BlockSpec((bB, d), lambda b, t: (b, 0))
    full_spec = lambda r, c: pl.BlockSpec((r, c), lambda b, t: (0, 0))

    out_tb, player_emb = pl.pallas_call(
        functools.partial(_mlp_body, D_model),
        grid=(nB, T),
        in_specs=[
            pl.BlockSpec((None, bB, D_base), lambda b, t: (t, b, 0)),
            row_spec(D_id),
            row_spec(D_id),
            row_spec(D_pos),
            row_spec(D_seq),
            row_spec(1),
            full_spec(D_base, D_model),
            full_spec(D_id, D_model),
            full_spec(D_id, D_model),
            full_spec(D_pos, D_model),
            full_spec(D_seq, D_model),
            full_spec(1, D_model),
            full_spec(1, D_model),
            full_spec(1, D_model),
            full_spec(D_model, D_model),
            full_spec(1, D_model),
        ],
        out_specs=[
            pl.BlockSpec((None, bB, D_model), lambda b, t: (t, b, 0)),
            row_spec(D_id),
        ],
        out_shape=[
            jax.ShapeDtypeStruct((T, B, D_model), jnp.float32),
            jax.ShapeDtypeStruct((B, D_id), jnp.float32),
        ],
        scratch_shapes=[pltpu.VMEM((bB, D_model), jnp.float32)],
        compiler_params=pltpu.CompilerParams(
            dimension_semantics=("parallel", "arbitrary"),
        ),
    )(e_bt, player_raw, team_emb, pos_emb, seq_emb, fmask,
      W1a, W1p, W1t, W1pos, W1seq, b1r, gr, lbr, W2, b2r)

    return (jnp.transpose(out_tb, (1, 0, 2)), player_emb, team_emb)


# speedup vs baseline: 1.8376x; 1.1100x over previous
import functools

import jax
import jax.numpy as jnp
from jax import lax
from jax.experimental import pallas as pl
from jax.experimental.pallas import tpu as pltpu
from jax.experimental.pallas import tpu_sc as plsc


def _tc_player_gather(EpT, H_player, ENT_PER_STEP=8):
    D_id, N = EpT.shape
    B = H_player.shape[0]
    steps = B // ENT_PER_STEP

    def body(idx_ref, *refs):
        out_ref = refs[-1]
        i = pl.program_id(0)
        lane_iota = lax.broadcasted_iota(jnp.int32, (1, 128), 1)
        for j in range(ENT_PER_STEP):
            lane = lax.rem(idx_ref[i * ENT_PER_STEP + j], 128)
            onehot = (lane_iota == lane).astype(jnp.float32)
            row = lax.dot_general(onehot, refs[j][...], (((1,), (1,)), ((), ())),
                                  preferred_element_type=jnp.float32)
            out_ref[pl.ds(j, 1), :] = row

    def tab_spec(j):
        return pl.BlockSpec(
            (D_id, 128),
            lambda i, idx_ref, j=j: (0, idx_ref[i * ENT_PER_STEP + j] // 128))

    grid_spec = pltpu.PrefetchScalarGridSpec(
        num_scalar_prefetch=1,
        grid=(steps,),
        in_specs=[tab_spec(j) for j in range(ENT_PER_STEP)],
        out_specs=pl.BlockSpec((ENT_PER_STEP, D_id),
                               lambda i, idx_ref: (i, 0)),
    )
    return pl.pallas_call(
        body,
        grid_spec=grid_spec,
        out_shape=jax.ShapeDtypeStruct((B, D_id), jnp.float32),
        compiler_params=pltpu.CompilerParams(
            dimension_semantics=("arbitrary",),
        ),
    )(H_player, *([EpT] * ENT_PER_STEP))


def _sc_gather_call(E_team, H_team, E_pos, H_pos, E_seq, H_seq):
    B = H_team.shape[0]
    D_id = E_team.shape[1]
    D_pos = E_pos.shape[1]
    D_seq = E_seq.shape[1]

    info = plsc.get_sparse_core_info()
    NC, NS, L = info.num_cores, info.num_subcores, info.num_lanes
    NW = NC * NS
    b_per_w = B // NW
    n_chunks = b_per_w // L
    assert B % (L * NW) == 0

    mesh = plsc.VectorSubcoreMesh(core_axis_name="c", subcore_axis_name="s")

    @functools.partial(
        pl.kernel,
        mesh=mesh,
        out_type=[
            jax.ShapeDtypeStruct((B, D_id), jnp.float32),
            jax.ShapeDtypeStruct((B, D_pos), jnp.float32),
            jax.ShapeDtypeStruct((B, D_seq), jnp.float32),
        ],
        scratch_types=[
            pltpu.VMEM((b_per_w,), jnp.int32),
            pltpu.VMEM((b_per_w, D_id), jnp.float32),
            pltpu.VMEM((b_per_w, D_pos), jnp.float32),
            pltpu.VMEM((b_per_w, D_seq), jnp.float32),
            pltpu.SemaphoreType.DMA,
        ],
    )
    def gather_kernel(et, ht, epos, hpos, eseq, hseq,
                      out_t, out_pos, out_seq,
                      iv, rt, rpos, rseq, sem):
        wid = lax.axis_index("s") * NC + lax.axis_index("c")
        sl = pl.ds(wid * b_per_w, b_per_w)

        def enqueue_rows(h_hbm, tab_hbm, rbuf):
            pltpu.sync_copy(h_hbm.at[sl], iv)

            def chunk_body(c, _):
                base = c * L
                chunk = iv[pl.ds(base, L)]
                for j in range(L):
                    pltpu.async_copy(tab_hbm.at[pl.ds(chunk[j], 1)],
                                     rbuf.at[pl.ds(base + j, 1)], sem)
                return ()

            lax.fori_loop(0, n_chunks, chunk_body, ())

        enqueue_rows(ht, et, rt)
        enqueue_rows(hpos, epos, rpos)
        enqueue_rows(hseq, eseq, rseq)
        pltpu.make_async_copy(et.at[pl.ds(0, b_per_w)], rt, sem).wait()
        pltpu.make_async_copy(epos.at[pl.ds(0, b_per_w)], rpos, sem).wait()
        pltpu.make_async_copy(eseq.at[pl.ds(0, b_per_w)], rseq, sem).wait()
        pltpu.sync_copy(rt, out_t.at[sl])
        pltpu.sync_copy(rpos, out_pos.at[sl])
        pltpu.sync_copy(rseq, out_seq.at[sl])

    return gather_kernel(E_team, H_team, E_pos, H_pos, E_seq, H_seq)


def _mlp_body(D_model,
              eb, praw, temb, posemb, seqemb, fm,
              w1a, w1p, w1t, w1pos, w1seq, b1, g, lb, w2, b2,
              out, pout, hrow_ref):
    f32 = jnp.float32
    t = pl.program_id(1)

    @pl.when(t == 0)
    def _():
        keep = 1.0 - fm[...]
        pm = praw[...] * keep
        pout[...] = pm
        hrow_ref[...] = (
            jnp.dot(pm, w1p[...], preferred_element_type=f32)
            + jnp.dot(temb[...], w1t[...], preferred_element_type=f32)
            + jnp.dot(posemb[...], w1pos[...], preferred_element_type=f32)
            + jnp.dot(seqemb[...], w1seq[...], preferred_element_type=f32)
            + b1[...])

    bf16 = jnp.bfloat16
    x = eb[...].astype(bf16)
    h = jnp.dot(x, w1a[...].astype(bf16), preferred_element_type=f32) + hrow_ref[...]
    h = jnp.maximum(h, 0.0)
    inv_d = 1.0 / D_model
    mu = jnp.sum(h, axis=1, keepdims=True) * inv_d
    d = h - mu
    var = jnp.sum(d * d, axis=1, keepdims=True) * inv_d
    hn = d * lax.rsqrt(var + 1e-5) * g[...] + lb[...]
    out[...] = jnp.dot(hn.astype(bf16), w2[...].astype(bf16),
                       preferred_element_type=f32) + b2[...]


def kernel(e_base, H_player, H_team, H_poshint, H_seqtype, mask_player_vec,
           E_player, E_team, E_pos, E_seq, W1, b1, ln_g, ln_b, W2, b2):
    B, T, D_base = e_base.shape
    D_id = E_player.shape[1]
    D_pos = E_pos.shape[1]
    D_seq = E_seq.shape[1]
    D_model = W2.shape[1]

    player_raw = _tc_player_gather(E_player.T, H_player)
    team_emb, pos_emb, seq_emb = _sc_gather_call(
        E_team, H_team, E_pos, jnp.maximum(H_poshint, 0), E_seq, H_seqtype)

    o0, o1, o2, o3 = D_base, D_base + D_id, D_base + 2 * D_id, D_base + 2 * D_id + D_pos
    W1a = W1[:o0]
    W1p = W1[o0:o1]
    W1t = W1[o1:o2]
    W1pos = W1[o2:o3]
    W1seq = W1[o3:]

    e_bt = jnp.transpose(e_base, (1, 0, 2))
    fmask = mask_player_vec.astype(jnp.float32).reshape(B, 1)
    b1r = b1.reshape(1, D_model)
    gr = ln_g.reshape(1, D_model)
    lbr = ln_b.reshape(1, D_model)
    b2r = b2.reshape(1, D_model)

    bB = 1024
    nB = B // bB

    row_spec = lambda d: pl.BlockSpec((bB, d), lambda b, t: (b, 0))
    full_spec = lambda r, c: pl.BlockSpec((r, c), lambda b, t: (0, 0))

    out_tb, player_emb = pl.pallas_call(
        functools.partial(_mlp_body, D_model),
        grid=(nB, T),
        in_specs=[
            pl.BlockSpec((None, bB, D_base), lambda b, t: (t, b, 0)),
            row_spec(D_id),
            row_spec(D_id),
            row_spec(D_pos),
            row_spec(D_seq),
            row_spec(1),
            full_spec(D_base, D_model),
            full_spec(D_id, D_model),
            full_spec(D_id, D_model),
            full_spec(D_pos, D_model),
            full_spec(D_seq, D_model),
            full_spec(1, D_model),
            full_spec(1, D_model),
            full_spec(1, D_model),
            full_spec(D_model, D_model),
            full_spec(1, D_model),
        ],
        out_specs=[
            pl.BlockSpec((None, bB, D_model), lambda b, t: (t, b, 0)),
            row_spec(D_id),
        ],
        out_shape=[
            jax.ShapeDtypeStruct((T, B, D_model), jnp.float32),
            jax.ShapeDtypeStruct((B, D_id), jnp.float32),
        ],
        scratch_shapes=[pltpu.VMEM((bB, D_model), jnp.float32)],
        compiler_params=pltpu.CompilerParams(
            dimension_semantics=("parallel", "arbitrary"),
        ),
    )(e_bt, player_raw, team_emb, pos_emb, seq_emb, fmask,
      W1a, W1p, W1t, W1pos, W1seq, b1r, gr, lbr, W2, b2r)

    return (jnp.transpose(out_tb, (1, 0, 2)), player_emb, team_emb)


# speedup vs baseline: 2.5334x; 1.3787x over previous
import functools

import jax
import jax.numpy as jnp
from jax import lax
from jax.experimental import pallas as pl
from jax.experimental.pallas import tpu as pltpu
from jax.experimental.pallas import tpu_sc as plsc


def _tc_player_gather(EpT, H_player, ENT_PER_STEP=32):
    D_id, N = EpT.shape
    B = H_player.shape[0]
    steps = B // ENT_PER_STEP

    def body(idx_ref, *refs):
        out_ref = refs[-1]
        i = pl.program_id(0)
        lane_iota = lax.broadcasted_iota(jnp.int32, (1, 128), 1)
        for j in range(ENT_PER_STEP):
            lane = lax.rem(idx_ref[i * ENT_PER_STEP + j], 128)
            onehot = (lane_iota == lane).astype(jnp.float32)
            row = lax.dot_general(onehot, refs[j][...], (((1,), (1,)), ((), ())),
                                  preferred_element_type=jnp.float32)
            out_ref[pl.ds(j, 1), :] = row

    def tab_spec(j):
        return pl.BlockSpec(
            (D_id, 128),
            lambda i, idx_ref, j=j: (0, idx_ref[i * ENT_PER_STEP + j] // 128))

    grid_spec = pltpu.PrefetchScalarGridSpec(
        num_scalar_prefetch=1,
        grid=(steps,),
        in_specs=[tab_spec(j) for j in range(ENT_PER_STEP)],
        out_specs=pl.BlockSpec((ENT_PER_STEP, D_id),
                               lambda i, idx_ref: (i, 0)),
    )
    return pl.pallas_call(
        body,
        grid_spec=grid_spec,
        out_shape=jax.ShapeDtypeStruct((B, D_id), jnp.float32),
        compiler_params=pltpu.CompilerParams(
            dimension_semantics=("arbitrary",),
        ),
    )(H_player, *([EpT] * ENT_PER_STEP))


def _sc_gather_call(E_team, H_team, E_pos, H_pos, E_seq, H_seq):
    B = H_team.shape[0]
    D_id = E_team.shape[1]
    D_pos = E_pos.shape[1]
    D_seq = E_seq.shape[1]

    info = plsc.get_sparse_core_info()
    NC, NS, L = info.num_cores, info.num_subcores, info.num_lanes
    NW = NC * NS
    b_per_w = B // NW
    n_chunks = b_per_w // L
    assert B % (L * NW) == 0

    mesh = plsc.VectorSubcoreMesh(core_axis_name="c", subcore_axis_name="s")

    @functools.partial(
        pl.kernel,
        mesh=mesh,
        out_type=[
            jax.ShapeDtypeStruct((B, D_id), jnp.float32),
            jax.ShapeDtypeStruct((B, D_pos), jnp.float32),
            jax.ShapeDtypeStruct((B, D_seq), jnp.float32),
        ],
        scratch_types=[
            pltpu.VMEM((b_per_w,), jnp.int32),
            pltpu.VMEM((b_per_w, D_id), jnp.float32),
            pltpu.VMEM((b_per_w, D_pos), jnp.float32),
            pltpu.VMEM((b_per_w, D_seq), jnp.float32),
            pltpu.SemaphoreType.DMA,
        ],
    )
    def gather_kernel(et, ht, epos, hpos, eseq, hseq,
                      out_t, out_pos, out_seq,
                      iv, rt, rpos, rseq, sem):
        wid = lax.axis_index("s") * NC + lax.axis_index("c")
        sl = pl.ds(wid * b_per_w, b_per_w)

        def enqueue_rows(h_hbm, tab_hbm, rbuf):
            pltpu.sync_copy(h_hbm.at[sl], iv)

            def chunk_body(c, _):
                base = c * L
                chunk = iv[pl.ds(base, L)]
                for j in range(L):
                    pltpu.async_copy(tab_hbm.at[pl.ds(chunk[j], 1)],
                                     rbuf.at[pl.ds(base + j, 1)], sem)
                return ()

            lax.fori_loop(0, n_chunks, chunk_body, ())

        enqueue_rows(ht, et, rt)
        enqueue_rows(hpos, epos, rpos)
        enqueue_rows(hseq, eseq, rseq)
        pltpu.make_async_copy(et.at[pl.ds(0, b_per_w)], rt, sem).wait()
        pltpu.make_async_copy(epos.at[pl.ds(0, b_per_w)], rpos, sem).wait()
        pltpu.make_async_copy(eseq.at[pl.ds(0, b_per_w)], rseq, sem).wait()
        pltpu.sync_copy(rt, out_t.at[sl])
        pltpu.sync_copy(rpos, out_pos.at[sl])
        pltpu.sync_copy(rseq, out_seq.at[sl])

    return gather_kernel(E_team, H_team, E_pos, H_pos, E_seq, H_seq)


def _mlp_body(D_model,
              eb, praw, temb, posemb, seqemb, fm,
              w1a, w1p, w1t, w1pos, w1seq, b1, g, lb, w2, b2,
              out, pout, hrow_ref):
    f32 = jnp.float32
    t = pl.program_id(1)

    @pl.when(t == 0)
    def _():
        keep = 1.0 - fm[...]
        pm = praw[...] * keep
        pout[...] = pm
        hrow_ref[...] = (
            jnp.dot(pm, w1p[...], preferred_element_type=f32)
            + jnp.dot(temb[...], w1t[...], preferred_element_type=f32)
            + jnp.dot(posemb[...], w1pos[...], preferred_element_type=f32)
            + jnp.dot(seqemb[...], w1seq[...], preferred_element_type=f32)
            + b1[...])

    bf16 = jnp.bfloat16
    x = eb[...].astype(bf16)
    h = jnp.dot(x, w1a[...].astype(bf16), preferred_element_type=f32) + hrow_ref[...]
    h = jnp.maximum(h, 0.0)
    inv_d = 1.0 / D_model
    mu = jnp.sum(h, axis=1, keepdims=True) * inv_d
    d = h - mu
    var = jnp.sum(d * d, axis=1, keepdims=True) * inv_d
    hn = d * lax.rsqrt(var + 1e-5) * g[...] + lb[...]
    out[...] = jnp.dot(hn.astype(bf16), w2[...].astype(bf16),
                       preferred_element_type=f32) + b2[...]


def kernel(e_base, H_player, H_team, H_poshint, H_seqtype, mask_player_vec,
           E_player, E_team, E_pos, E_seq, W1, b1, ln_g, ln_b, W2, b2):
    B, T, D_base = e_base.shape
    D_id = E_player.shape[1]
    D_pos = E_pos.shape[1]
    D_seq = E_seq.shape[1]
    D_model = W2.shape[1]

    player_raw = _tc_player_gather(E_player.T, H_player)
    team_emb, pos_emb, seq_emb = _sc_gather_call(
        E_team, H_team, E_pos, jnp.maximum(H_poshint, 0), E_seq, H_seqtype)

    o0, o1, o2, o3 = D_base, D_base + D_id, D_base + 2 * D_id, D_base + 2 * D_id + D_pos
    W1a = W1[:o0]
    W1p = W1[o0:o1]
    W1t = W1[o1:o2]
    W1pos = W1[o2:o3]
    W1seq = W1[o3:]

    e_bt = jnp.transpose(e_base, (1, 0, 2))
    fmask = mask_player_vec.astype(jnp.float32).reshape(B, 1)
    b1r = b1.reshape(1, D_model)
    gr = ln_g.reshape(1, D_model)
    lbr = ln_b.reshape(1, D_model)
    b2r = b2.reshape(1, D_model)

    bB = 1024
    nB = B // bB

    row_spec = lambda d: pl.BlockSpec((bB, d), lambda b, t: (b, 0))
    full_spec = lambda r, c: pl.BlockSpec((r, c), lambda b, t: (0, 0))

    out_tb, player_emb = pl.pallas_call(
        functools.partial(_mlp_body, D_model),
        grid=(nB, T),
        in_specs=[
            pl.BlockSpec((None, bB, D_base), lambda b, t: (t, b, 0)),
            row_spec(D_id),
            row_spec(D_id),
            row_spec(D_pos),
            row_spec(D_seq),
            row_spec(1),
            full_spec(D_base, D_model),
            full_spec(D_id, D_model),
            full_spec(D_id, D_model),
            full_spec(D_pos, D_model),
            full_spec(D_seq, D_model),
            full_spec(1, D_model),
            full_spec(1, D_model),
            full_spec(1, D_model),
            full_spec(D_model, D_model),
            full_spec(1, D_model),
        ],
        out_specs=[
            pl.BlockSpec((None, bB, D_model), lambda b, t: (t, b, 0)),
            row_spec(D_id),
        ],
        out_shape=[
            jax.ShapeDtypeStruct((T, B, D_model), jnp.float32),
            jax.ShapeDtypeStruct((B, D_id), jnp.float32),
        ],
        scratch_shapes=[pltpu.VMEM((bB, D_model), jnp.float32)],
        compiler_params=pltpu.CompilerParams(
            dimension_semantics=("parallel", "arbitrary"),
        ),
    )(e_bt, player_raw, team_emb, pos_emb, seq_emb, fmask,
      W1a, W1p, W1t, W1pos, W1seq, b1r, gr, lbr, W2, b2r)

    return (jnp.transpose(out_tb, (1, 0, 2)), player_emb, team_emb)


# speedup vs baseline: 2.5803x; 1.0185x over previous
import functools

import jax
import jax.numpy as jnp
from jax import lax
from jax.experimental import pallas as pl
from jax.experimental.pallas import tpu as pltpu
from jax.experimental.pallas import tpu_sc as plsc


def _tc_player_gather(EpT, H_player, ENT_PER_STEP=64):
    D_id, N = EpT.shape
    B = H_player.shape[0]
    steps = B // ENT_PER_STEP

    def body(idx_ref, *refs):
        out_ref = refs[-1]
        i = pl.program_id(0)
        lane_iota = lax.broadcasted_iota(jnp.int32, (1, 128), 1)
        for j in range(ENT_PER_STEP):
            lane = lax.rem(idx_ref[i * ENT_PER_STEP + j], 128)
            onehot = (lane_iota == lane).astype(jnp.float32)
            row = lax.dot_general(onehot, refs[j][...], (((1,), (1,)), ((), ())),
                                  preferred_element_type=jnp.float32)
            out_ref[pl.ds(j, 1), :] = row

    def tab_spec(j):
        return pl.BlockSpec(
            (D_id, 128),
            lambda i, idx_ref, j=j: (0, idx_ref[i * ENT_PER_STEP + j] // 128))

    grid_spec = pltpu.PrefetchScalarGridSpec(
        num_scalar_prefetch=1,
        grid=(steps,),
        in_specs=[tab_spec(j) for j in range(ENT_PER_STEP)],
        out_specs=pl.BlockSpec((ENT_PER_STEP, D_id),
                               lambda i, idx_ref: (i, 0)),
    )
    return pl.pallas_call(
        body,
        grid_spec=grid_spec,
        out_shape=jax.ShapeDtypeStruct((B, D_id), jnp.float32),
        compiler_params=pltpu.CompilerParams(
            dimension_semantics=("arbitrary",),
        ),
    )(H_player, *([EpT] * ENT_PER_STEP))


def _sc_gather_call(E_team, H_team, E_pos, H_pos, E_seq, H_seq):
    B = H_team.shape[0]
    D_id = E_team.shape[1]
    D_pos = E_pos.shape[1]
    D_seq = E_seq.shape[1]

    info = plsc.get_sparse_core_info()
    NC, NS, L = info.num_cores, info.num_subcores, info.num_lanes
    NW = NC * NS
    b_per_w = B // NW
    n_chunks = b_per_w // L
    assert B % (L * NW) == 0

    mesh = plsc.VectorSubcoreMesh(core_axis_name="c", subcore_axis_name="s")

    @functools.partial(
        pl.kernel,
        mesh=mesh,
        out_type=[
            jax.ShapeDtypeStruct((B, D_id), jnp.float32),
            jax.ShapeDtypeStruct((B, D_pos), jnp.float32),
            jax.ShapeDtypeStruct((B, D_seq), jnp.float32),
        ],
        scratch_types=[
            pltpu.VMEM((b_per_w,), jnp.int32),
            pltpu.VMEM((b_per_w, D_id), jnp.float32),
            pltpu.VMEM((b_per_w, D_pos), jnp.float32),
            pltpu.VMEM((b_per_w, D_seq), jnp.float32),
            pltpu.SemaphoreType.DMA,
        ],
    )
    def gather_kernel(et, ht, epos, hpos, eseq, hseq,
                      out_t, out_pos, out_seq,
                      iv, rt, rpos, rseq, sem):
        wid = lax.axis_index("s") * NC + lax.axis_index("c")
        sl = pl.ds(wid * b_per_w, b_per_w)

        def enqueue_rows(h_hbm, tab_hbm, rbuf):
            pltpu.sync_copy(h_hbm.at[sl], iv)

            def chunk_body(c, _):
                base = c * L
                chunk = iv[pl.ds(base, L)]
                for j in range(L):
                    pltpu.async_copy(tab_hbm.at[pl.ds(chunk[j], 1)],
                                     rbuf.at[pl.ds(base + j, 1)], sem)
                return ()

            lax.fori_loop(0, n_chunks, chunk_body, ())

        enqueue_rows(ht, et, rt)
        enqueue_rows(hpos, epos, rpos)
        enqueue_rows(hseq, eseq, rseq)
        pltpu.make_async_copy(et.at[pl.ds(0, b_per_w)], rt, sem).wait()
        pltpu.make_async_copy(epos.at[pl.ds(0, b_per_w)], rpos, sem).wait()
        pltpu.make_async_copy(eseq.at[pl.ds(0, b_per_w)], rseq, sem).wait()
        pltpu.sync_copy(rt, out_t.at[sl])
        pltpu.sync_copy(rpos, out_pos.at[sl])
        pltpu.sync_copy(rseq, out_seq.at[sl])

    return gather_kernel(E_team, H_team, E_pos, H_pos, E_seq, H_seq)


def _mlp_body(D_model,
              eb, praw, temb, posemb, seqemb, fm,
              w1a, w1p, w1t, w1pos, w1seq, b1, g, lb, w2, b2,
              out, pout, hrow_ref):
    f32 = jnp.float32
    t = pl.program_id(1)

    @pl.when(t == 0)
    def _():
        keep = 1.0 - fm[...]
        pm = praw[...] * keep
        pout[...] = pm
        hrow_ref[...] = (
            jnp.dot(pm, w1p[...], preferred_element_type=f32)
            + jnp.dot(temb[...], w1t[...], preferred_element_type=f32)
            + jnp.dot(posemb[...], w1pos[...], preferred_element_type=f32)
            + jnp.dot(seqemb[...], w1seq[...], preferred_element_type=f32)
            + b1[...])

    bf16 = jnp.bfloat16
    x = eb[...].astype(bf16)
    h = jnp.dot(x, w1a[...].astype(bf16), preferred_element_type=f32) + hrow_ref[...]
    h = jnp.maximum(h, 0.0)
    inv_d = 1.0 / D_model
    mu = jnp.sum(h, axis=1, keepdims=True) * inv_d
    d = h - mu
    var = jnp.sum(d * d, axis=1, keepdims=True) * inv_d
    hn = d * lax.rsqrt(var + 1e-5) * g[...] + lb[...]
    out[...] = jnp.dot(hn.astype(bf16), w2[...].astype(bf16),
                       preferred_element_type=f32) + b2[...]


def kernel(e_base, H_player, H_team, H_poshint, H_seqtype, mask_player_vec,
           E_player, E_team, E_pos, E_seq, W1, b1, ln_g, ln_b, W2, b2):
    B, T, D_base = e_base.shape
    D_id = E_player.shape[1]
    D_pos = E_pos.shape[1]
    D_seq = E_seq.shape[1]
    D_model = W2.shape[1]

    player_raw = _tc_player_gather(E_player.T, H_player)
    team_emb, pos_emb, seq_emb = _sc_gather_call(
        E_team, H_team, E_pos, jnp.maximum(H_poshint, 0), E_seq, H_seqtype)

    o0, o1, o2, o3 = D_base, D_base + D_id, D_base + 2 * D_id, D_base + 2 * D_id + D_pos
    W1a = W1[:o0]
    W1p = W1[o0:o1]
    W1t = W1[o1:o2]
    W1pos = W1[o2:o3]
    W1seq = W1[o3:]

    e_bt = jnp.transpose(e_base, (1, 0, 2))
    fmask = mask_player_vec.astype(jnp.float32).reshape(B, 1)
    b1r = b1.reshape(1, D_model)
    gr = ln_g.reshape(1, D_model)
    lbr = ln_b.reshape(1, D_model)
    b2r = b2.reshape(1, D_model)

    bB = 1024
    nB = B // bB

    row_spec = lambda d: pl.BlockSpec((bB, d), lambda b, t: (b, 0))
    full_spec = lambda r, c: pl.BlockSpec((r, c), lambda b, t: (0, 0))

    out_tb, player_emb = pl.pallas_call(
        functools.partial(_mlp_body, D_model),
        grid=(nB, T),
        in_specs=[
            pl.BlockSpec((None, bB, D_base), lambda b, t: (t, b, 0)),
            row_spec(D_id),
            row_spec(D_id),
            row_spec(D_pos),
            row_spec(D_seq),
            row_spec(1),
            full_spec(D_base, D_model),
            full_spec(D_id, D_model),
            full_spec(D_id, D_model),
            full_spec(D_pos, D_model),
            full_spec(D_seq, D_model),
            full_spec(1, D_model),
            full_spec(1, D_model),
            full_spec(1, D_model),
            full_spec(D_model, D_model),
            full_spec(1, D_model),
        ],
        out_specs=[
            pl.BlockSpec((None, bB, D_model), lambda b, t: (t, b, 0)),
            row_spec(D_id),
        ],
        out_shape=[
            jax.ShapeDtypeStruct((T, B, D_model), jnp.float32),
            jax.ShapeDtypeStruct((B, D_id), jnp.float32),
        ],
        scratch_shapes=[pltpu.VMEM((bB, D_model), jnp.float32)],
        compiler_params=pltpu.CompilerParams(
            dimension_semantics=("parallel", "arbitrary"),
        ),
    )(e_bt, player_raw, team_emb, pos_emb, seq_emb, fmask,
      W1a, W1p, W1t, W1pos, W1seq, b1r, gr, lbr, W2, b2r)

    return (jnp.transpose(out_tb, (1, 0, 2)), player_emb, team_emb)
